# Initial kernel scaffold; baseline (speedup 1.0000x reference)
import jax
import jax.numpy as jnp
from jax.experimental import pallas as pl


def kernel(x, edge_index, batch, node_type, params):
    raise NotImplementedError("write your pallas kernel here")


# speedup vs baseline: 7.2686x; 7.2686x over previous
import functools

import jax
import jax.numpy as jnp
from jax import lax
from jax.experimental import pallas as pl
from jax.experimental.pallas import tpu as pltpu
from jax.experimental.pallas import tpu_sc as plsc

N = 10000
E = 160000
B = 16
NODE_DIM = 1310
LIG_DIM = 36
H = 256
ATT = 128
HEADS = 4
LAYERS = 3

NPAD = 10240
TILES = 16
CH = 64
CHUNKS = 160
SUP = 16
PER_TILE = E // TILES
HALF = 128
BN = 400
GRID = N // BN

_mesh = plsc.VectorSubcoreMesh(
    core_axis_name="c", subcore_axis_name="s", num_cores=2, num_subcores=16)

_DOT = dict(precision=lax.Precision.DEFAULT, preferred_element_type=jnp.float32)


@functools.partial(
    pl.kernel, mesh=_mesh,
    out_type=jax.ShapeDtypeStruct((2, NPAD, HALF), jnp.float32),
    scratch_types=[
        pltpu.VMEM((CHUNKS, CH), jnp.int32),
        pltpu.VMEM((CH, HALF), jnp.float32),
        pltpu.VMEM((64, HALF), jnp.float32),
        pltpu.VMEM_SHARED((NPAD, HALF), jnp.float32),
    ],
)
def _deg_kernel(dstp_hbm, ones_hbm, zeros_hbm, deg_hbm,
                idx_d, ones_v, zero_v, acc):
    c = lax.axis_index("c")
    t = lax.axis_index("s")
    pltpu.sync_copy(dstp_hbm.at[t], idx_d)
    pltpu.sync_copy(ones_hbm, ones_v)
    pltpu.sync_copy(zeros_hbm, zero_v)

    def zero_stripe(k, _):
        pltpu.sync_copy(zero_v, acc.at[pl.ds(t * 640 + k * 64, 64)])
        return 0
    lax.fori_loop(0, 10, zero_stripe, 0)
    plsc.subcore_barrier()

    def body(j, _):
        pltpu.sync_copy(ones_v, acc.at[idx_d.at[c * (CHUNKS // 2) + j]], add=True)
        return 0
    lax.fori_loop(0, CHUNKS // 2, body, 0)
    plsc.subcore_barrier()

    pltpu.sync_copy(acc.at[pl.ds(t * 640, 640)],
                    deg_hbm.at[c, pl.ds(t * 640, 640)])


@functools.partial(
    pl.kernel, mesh=_mesh,
    out_type=(jax.ShapeDtypeStruct((NPAD, HALF), jnp.float32),
              jax.ShapeDtypeStruct((NPAD, HALF), jnp.float32)),
    scratch_types=[
        pltpu.VMEM((SUP, CH), jnp.int32),
        pltpu.VMEM((SUP, CH), jnp.int32),
        pltpu.VMEM((SUP, CH), jnp.int32),
        pltpu.VMEM((SUP, CH), jnp.int32),
        pltpu.VMEM((CH, HALF), jnp.float32),
        pltpu.VMEM((CH, HALF), jnp.float32),
        pltpu.VMEM_SHARED((NPAD, HALF), jnp.float32),
        pltpu.SemaphoreType.DMA,
        pltpu.SemaphoreType.DMA,
        pltpu.SemaphoreType.DMA,
    ],
)
def _scatter_kernel(ya_hbm, yb_hbm, srcp_hbm, dstp_hbm, zeros_hbm,
                    sa_hbm, sb_hbm,
                    is0, id0, is1, id1, buf0, buf1, acc, sem0, sem1, semi):
    c = lax.axis_index("c")
    t = lax.axis_index("s")
    pltpu.sync_copy(zeros_hbm, buf0)

    def zero_stripe(k, _):
        pltpu.sync_copy(buf0, acc.at[pl.ds(t * 640 + k * 64, 64)])
        return 0
    lax.fori_loop(0, 10, zero_stripe, 0)
    plsc.subcore_barrier()

    def run(y_hbm, out_hbm):
        pltpu.sync_copy(srcp_hbm.at[t, pl.ds(0, SUP)], is0)
        pltpu.sync_copy(dstp_hbm.at[t, pl.ds(0, SUP)], id0)
        for g in range(CHUNKS // SUP):
            cs, cd = (is0, id0) if g % 2 == 0 else (is1, id1)
            ns, nd = (is1, id1) if g % 2 == 0 else (is0, id0)
            if g > 0:
                pltpu.make_async_copy(
                    srcp_hbm.at[t, pl.ds(g * SUP, SUP)], cs, semi).wait()
                pltpu.make_async_copy(
                    dstp_hbm.at[t, pl.ds(g * SUP, SUP)], cd, semi).wait()
            if g + 1 < CHUNKS // SUP:
                pltpu.async_copy(
                    srcp_hbm.at[t, pl.ds((g + 1) * SUP, SUP)], ns, semi)
                pltpu.async_copy(
                    dstp_hbm.at[t, pl.ds((g + 1) * SUP, SUP)], nd, semi)
            pltpu.async_copy(y_hbm.at[cs.at[0]], buf0, sem0)
            pltpu.async_copy(y_hbm.at[cs.at[1]], buf1, sem1)

            def body(i, _):
                pltpu.make_async_copy(y_hbm.at[cs.at[2 * i]], buf0, sem0).wait()
                pltpu.sync_copy(buf0, acc.at[cd.at[2 * i]], add=True)

                @pl.when(2 * i + 2 < SUP)
                def _():
                    pltpu.async_copy(y_hbm.at[cs.at[2 * i + 2]], buf0, sem0)

                pltpu.make_async_copy(y_hbm.at[cs.at[2 * i + 1]], buf1, sem1).wait()
                pltpu.sync_copy(buf1, acc.at[cd.at[2 * i + 1]], add=True)

                @pl.when(2 * i + 3 < SUP)
                def _():
                    pltpu.async_copy(y_hbm.at[cs.at[2 * i + 3]], buf1, sem1)
                return 0
            lax.fori_loop(0, SUP // 2, body, 0)
        plsc.subcore_barrier()
        pltpu.sync_copy(acc.at[pl.ds(t * 640, 640)],
                        out_hbm.at[pl.ds(t * 640, 640)])

    @pl.when(c == 0)
    def _():
        run(ya_hbm, sa_hbm)

    @pl.when(c == 1)
    def _():
        run(yb_hbm, sb_hbm)


def _ln(x, g, b):
    mu = jnp.mean(x, axis=-1, keepdims=True)
    var = jnp.mean((x - mu) ** 2, axis=-1, keepdims=True)
    return (x - mu) * lax.rsqrt(var + 1e-5) * g + b


def _dis_of(dega_ref, degb_ref):
    deg = dega_ref[...][:, 0:1] + degb_ref[...][:, 0:1] + 1.0
    return lax.rsqrt(deg)


def _dotT(a, b):
    return lax.dot_general(a, b, (((0,), (0,)), ((), ())),
                           precision=lax.Precision.DEFAULT,
                           preferred_element_type=jnp.float32)


def _k1_body(x_ref, nt_ref, dega_ref, degb_ref,
             Wp_ref, bp_ref, gp_ref, bep_ref,
             Wl_ref, bl_ref, gl_ref, bel_ref,
             emb_ref, W0_ref, ya_ref, yb_ref):
    xb = x_ref[...]
    hp = jnp.maximum(_ln(jnp.dot(xb, Wp_ref[...], **_DOT) + bp_ref[...],
                         gp_ref[...], bep_ref[...]), 0.0)
    hl = jnp.maximum(_ln(jnp.dot(xb[:, :40], Wl_ref[...], **_DOT) + bl_ref[...],
                         gl_ref[...], bel_ref[...]), 0.0)
    ispf = jnp.where(nt_ref[...] == 0, 1.0, 0.0)
    h = (hp * ispf + hl * (1.0 - ispf)
         + emb_ref[...][0:1, :] * ispf + emb_ref[...][1:2, :] * (1.0 - ispf))
    y = jnp.dot(h, W0_ref[...], **_DOT) * _dis_of(dega_ref, degb_ref)
    ya_ref[...] = y[:, :HALF]
    yb_ref[...] = y[:, HALF:]


def _k2_body(residual, *refs):
    if residual:
        (sa_ref, sb_ref, ya_ref, yb_ref, hprev_ref, dega_ref, degb_ref,
         gcnb_ref, lng_ref, lnb_ref, Wn_ref,
         h_ref, yna_ref, ynb_ref) = refs
    else:
        (sa_ref, sb_ref, ya_ref, yb_ref, dega_ref, degb_ref,
         gcnb_ref, lng_ref, lnb_ref, Wn_ref,
         h_ref, yna_ref, ynb_ref) = refs
    dis = _dis_of(dega_ref, degb_ref)
    s = jnp.concatenate([sa_ref[...], sb_ref[...]], axis=1)
    y = jnp.concatenate([ya_ref[...], yb_ref[...]], axis=1)
    xn = (s + y) * dis + gcnb_ref[...]
    if residual:
        xn = xn + hprev_ref[...]
    h = jnp.maximum(_ln(xn, lng_ref[...], lnb_ref[...]), 0.0)
    h_ref[...] = h
    yn = jnp.dot(h, Wn_ref[...], **_DOT) * dis
    yna_ref[...] = yn[:, :HALF]
    ynb_ref[...] = yn[:, HALF:]


def _k3_body(sa_ref, sb_ref, ya_ref, yb_ref, hprev_ref, dega_ref, degb_ref,
             gcnb_ref, lng_ref, lnb_ref,
             W1_ref, b1_ref, W2_ref, b2_ref, bt_ref, nt_ref,
             M_ref, sc_ref, smaxp_ref, smaxa_ref, pcr_ref, cntc_ref,
             smaxp_s, smaxa_s, pcr_s, cntc_s):
    j = pl.program_id(0)

    @pl.when(j == 0)
    def _():
        smaxp_s[...] = jnp.full((HEADS, B), -1e30, jnp.float32)
        smaxa_s[...] = jnp.full((HEADS, B), -1e30, jnp.float32)
        pcr_s[...] = jnp.zeros((1, B), jnp.float32)
        cntc_s[...] = jnp.zeros((B, 1), jnp.float32)

    dis = _dis_of(dega_ref, degb_ref)
    s = jnp.concatenate([sa_ref[...], sb_ref[...]], axis=1)
    y = jnp.concatenate([ya_ref[...], yb_ref[...]], axis=1)
    xn = (s + y) * dis + gcnb_ref[...] + hprev_ref[...]
    M = jnp.maximum(_ln(xn, lng_ref[...], lnb_ref[...]), 0.0)
    M_ref[...] = M
    sc = jnp.dot(jnp.tanh(jnp.dot(M, W1_ref[...], **_DOT) + b1_ref[...]),
                 W2_ref[...], **_DOT) + b2_ref[...]
    sc_ref[...] = sc

    protf = jnp.where(nt_ref[...] == 0, 1.0, 0.0)
    onehot = bt_ref[...] == lax.broadcasted_iota(jnp.int32, (BN, B), 1)
    of = jnp.where(onehot, 1.0, 0.0)
    ofp = of * protf
    pcr_s[...] += jnp.sum(ofp, axis=0, keepdims=True)
    cntc_s[...] += _dotT(of, jnp.ones((BN, 1), jnp.float32))
    for k in range(HEADS):
        sck = sc[:, k:k + 1]
        mp = jnp.max(jnp.where(ofp > 0.5, sck, -1e30), axis=0, keepdims=True)
        ma = jnp.max(jnp.where(onehot, sck, -1e30), axis=0, keepdims=True)
        smaxp_s[k:k + 1, :] = jnp.maximum(smaxp_s[k:k + 1, :], mp)
        smaxa_s[k:k + 1, :] = jnp.maximum(smaxa_s[k:k + 1, :], ma)

    @pl.when(j == GRID - 1)
    def _():
        smaxp_ref[...] = smaxp_s[...]
        smaxa_ref[...] = smaxa_s[...]
        pcr_ref[...] = pcr_s[...]
        cntc_ref[...] = cntc_s[...]


def _k4_body(M_ref, sc_ref, bt_ref, nt_ref, smaxp_ref, smaxa_ref,
             pcr_ref, cntc_ref,
             C1_ref, c1_ref, C2_ref, c2_ref, C3_ref, c3_ref,
             out_ref, P_s, denom_s, G_s):
    j = pl.program_id(0)

    @pl.when(j == 0)
    def _():
        P_s[...] = jnp.zeros((HEADS * B, H), jnp.float32)
        denom_s[...] = jnp.zeros((B, HEADS), jnp.float32)
        G_s[...] = jnp.zeros((B, H), jnp.float32)

    M = M_ref[...]
    sc = sc_ref[...]
    protf = jnp.where(nt_ref[...] == 0, 1.0, 0.0)
    notlig2 = jnp.where(nt_ref[...] != 1, 1.0, 0.0)
    onehot = bt_ref[...] == lax.broadcasted_iota(jnp.int32, (BN, B), 1)
    of = jnp.where(onehot, 1.0, 0.0)
    pcr = pcr_ref[...]

    hprow = jnp.sum(of * pcr, axis=1, keepdims=True)
    efff = jnp.where(hprow > 0, protf, 1.0)

    for k in range(HEADS):
        smaxsel = jnp.where(pcr > 0, smaxp_ref[...][k:k + 1, :],
                            smaxa_ref[...][k:k + 1, :])
        smaxrow = jnp.sum(of * smaxsel, axis=1, keepdims=True)
        ek = jnp.exp(sc[:, k:k + 1] - smaxrow) * efff
        denom_s[:, k:k + 1] += _dotT(of, ek)
        P_s[k * B:(k + 1) * B] += _dotT(of * ek, M)
    G_s[...] += _dotT(of * notlig2, M)

    @pl.when(j == GRID - 1)
    def _():
        att = jnp.zeros((B, H), jnp.float32)
        for k in range(HEADS):
            dk = jnp.maximum(denom_s[...][:, k:k + 1], 1e-30)
            att = att + P_s[k * B:(k + 1) * B] / dk
        att = att * (1.0 / HEADS)
        gpool = G_s[...] / jnp.maximum(cntc_ref[...], 1.0)
        g = jnp.concatenate([att, gpool], axis=1)
        z = jnp.maximum(jnp.dot(g, C1_ref[...], **_DOT) + c1_ref[...], 0.0)
        z = jnp.maximum(jnp.dot(z, C2_ref[...], **_DOT) + c2_ref[...], 0.0)
        out_ref[...] = jnp.dot(z, C3_ref[...], **_DOT) + c3_ref[...]


def _row_spec(w):
    return pl.BlockSpec((BN, w), lambda j: (j, 0))


def _full_spec(shape):
    nd = len(shape)
    return pl.BlockSpec(shape, lambda j: (0,) * nd)


def _sd(shape):
    return jax.ShapeDtypeStruct(shape, jnp.float32)


def kernel(x, edge_index, batch, node_type, params):
    src, dst = edge_index[0], edge_index[1]
    srcp = jnp.pad(src.reshape(TILES, PER_TILE), ((0, 0), (0, CHUNKS * CH - PER_TILE)),
                   constant_values=0).reshape(TILES, CHUNKS, CH)
    dstp = jnp.pad(dst.reshape(TILES, PER_TILE), ((0, 0), (0, CHUNKS * CH - PER_TILE)),
                   constant_values=N).reshape(TILES, CHUNKS, CH)
    nt2 = node_type.reshape(N, 1)
    bt2 = batch.reshape(N, 1)
    ones128 = jnp.ones((CH, HALF), jnp.float32)
    zeros128 = jnp.zeros((64, HALF), jnp.float32)

    p = params
    Wlp = jnp.pad(p['Wl'], ((0, 40 - LIG_DIM), (0, 0)))
    row = lambda v: v.reshape(1, -1)

    deg_f = _deg_kernel(dstp, ones128, zeros128)
    dega = deg_f[0, :, :16]
    degb = deg_f[1, :, :16]

    k1 = pl.pallas_call(
        _k1_body,
        grid=(GRID,),
        in_specs=[_row_spec(NODE_DIM), _row_spec(1), _row_spec(16), _row_spec(16),
                  _full_spec((NODE_DIM, H)), _full_spec((1, H)), _full_spec((1, H)),
                  _full_spec((1, H)),
                  _full_spec((40, H)), _full_spec((1, H)), _full_spec((1, H)),
                  _full_spec((1, H)),
                  _full_spec((2, H)), _full_spec((H, H))],
        out_specs=[_row_spec(HALF), _row_spec(HALF)],
        out_shape=[_sd((N, HALF)), _sd((N, HALF))],
    )
    ya0, yb0 = k1(x, nt2, dega, degb,
                  p['Wp'], row(p['bp']), row(p['gp']), row(p['bep']),
                  Wlp, row(p['bl']), row(p['gl']), row(p['bel']),
                  p['emb'], p['gcnW'][0])

    sa0, sb0 = _scatter_kernel(ya0, yb0, srcp, dstp, zeros128)

    def layer_call(residual, sa, sb, ya, yb, hprev, gcnb, lng, lnb, Wn):
        ins = [sa, sb, ya, yb] + ([hprev] if residual else []) + [dega, degb]
        ins += [row(gcnb), row(lng), row(lnb), Wn]
        specs = [_row_spec(HALF)] * 4 + ([_row_spec(H)] if residual else [])
        specs += [_row_spec(16), _row_spec(16),
                  _full_spec((1, H)), _full_spec((1, H)), _full_spec((1, H)),
                  _full_spec((H, H))]
        call = pl.pallas_call(
            functools.partial(_k2_body, residual),
            grid=(GRID,),
            in_specs=specs,
            out_specs=[_row_spec(H), _row_spec(HALF), _row_spec(HALF)],
            out_shape=[_sd((N, H)), _sd((N, HALF)), _sd((N, HALF))],
        )
        return call(*ins)

    h1, ya1, yb1 = layer_call(False, sa0, sb0, ya0, yb0, None,
                              p['gcnb'][0], p['lng'][0], p['lnb'][0], p['gcnW'][1])
    sa1, sb1 = _scatter_kernel(ya1, yb1, srcp, dstp, zeros128)

    h2, ya2, yb2 = layer_call(True, sa1, sb1, ya1, yb1, h1,
                              p['gcnb'][1], p['lng'][1], p['lnb'][1], p['gcnW'][2])
    sa2, sb2 = _scatter_kernel(ya2, yb2, srcp, dstp, zeros128)

    k3 = pl.pallas_call(
        _k3_body,
        grid=(GRID,),
        in_specs=[_row_spec(HALF)] * 4 + [_row_spec(H), _row_spec(16), _row_spec(16),
                  _full_spec((1, H)), _full_spec((1, H)), _full_spec((1, H)),
                  _full_spec((H, ATT)), _full_spec((1, ATT)),
                  _full_spec((ATT, HEADS)), _full_spec((1, HEADS)),
                  _row_spec(1), _row_spec(1)],
        out_specs=[_row_spec(H), _row_spec(HEADS),
                   _full_spec((HEADS, B)), _full_spec((HEADS, B)),
                   _full_spec((1, B)), _full_spec((B, 1))],
        out_shape=[_sd((N, H)), _sd((N, HEADS)),
                   _sd((HEADS, B)), _sd((HEADS, B)), _sd((1, B)), _sd((B, 1))],
        scratch_shapes=[pltpu.VMEM((HEADS, B), jnp.float32),
                        pltpu.VMEM((HEADS, B), jnp.float32),
                        pltpu.VMEM((1, B), jnp.float32),
                        pltpu.VMEM((B, 1), jnp.float32)],
    )
    M, sc, smaxp, smaxa, pcr, cntc = k3(
        sa2, sb2, ya2, yb2, h2, dega, degb,
        row(p['gcnb'][2]), row(p['lng'][2]), row(p['lnb'][2]),
        p['W1'], row(p['b1']), p['W2'], row(p['b2']), bt2, nt2)

    k4 = pl.pallas_call(
        _k4_body,
        grid=(GRID,),
        in_specs=[_row_spec(H), _row_spec(HEADS), _row_spec(1), _row_spec(1),
                  _full_spec((HEADS, B)), _full_spec((HEADS, B)),
                  _full_spec((1, B)), _full_spec((B, 1)),
                  _full_spec((2 * H, H)), _full_spec((1, H)),
                  _full_spec((H, ATT)), _full_spec((1, ATT)),
                  _full_spec((ATT, 2)), _full_spec((1, 2))],
        out_specs=[_full_spec((B, 2))],
        out_shape=[_sd((B, 2))],
        scratch_shapes=[pltpu.VMEM((HEADS * B, H), jnp.float32),
                        pltpu.VMEM((B, HEADS), jnp.float32),
                        pltpu.VMEM((B, H), jnp.float32)],
    )
    (out,) = k4(M, sc, bt2, nt2, smaxp, smaxa, pcr, cntc,
                p['C1'], row(p['c1']), p['C2'], row(p['c2']),
                p['C3'], row(p['c3']))
    return out
